# baseline (device time: 110934 ns/iter reference)
import jax
import jax.numpy as jnp
from jax import lax
from jax.experimental import pallas as pl
from jax.experimental.pallas import tpu as pltpu

_NB = 8
_PANELS = 2
_NG = _NB * _PANELS


def _fused_body(
    dy_hbm, w_hbm, out_hbm, dy_v, w_buf, p_buf, pb16, red32, comm, bcomm,
    ld_sems, st_sems, st2_sems, a_send, a_recv, b_send, b_recv,
):
    my_x = lax.axis_index("x")
    my_y = lax.axis_index("y")
    band = dy_v.shape[0]
    pm = band // _PANELS
    n_out = out_hbm.shape[1]
    cb = n_out // _NB
    row0 = my_y * band
    nbr_row0 = (1 - my_y) * band

    def w_cp(c):
        return pltpu.make_async_copy(
            w_hbm.at[pl.ds(c * cb, cb), :], w_buf.at[c % 2], ld_sems.at[c % 2]
        )

    def dy_cp(h):
        return pltpu.make_async_copy(
            dy_hbm.at[pl.ds(row0 + h * pm, pm), :],
            dy_v.at[pl.ds(h * pm, pm), :],
            ld_sems.at[2 + h],
        )

    w_cp(0).start()
    dy_cp(0).start()
    dy_cp(1).start()

    barrier = pltpu.get_barrier_semaphore()
    for nbr in ((1 - my_x, my_y), (my_x, 1 - my_y)):
        pl.semaphore_signal(
            barrier, inc=1, device_id=nbr, device_id_type=pl.DeviceIdType.MESH
        )
    pl.semaphore_wait(barrier, 2)

    def chunk_slice(g, base_row):
        return (
            pl.ds(base_row + (g % _PANELS) * pm, pm),
            pl.ds((g // _PANELS) * cb, cb),
        )

    def rdma_a(g):
        return pltpu.make_async_remote_copy(
            src_ref=pb16.at[g % 3],
            dst_ref=comm.at[g],
            send_sem=a_send.at[g % 3],
            recv_sem=a_recv.at[g],
            device_id=(1 - my_x, my_y),
            device_id_type=pl.DeviceIdType.MESH,
        )

    def rdma_b(g):
        return pltpu.make_async_remote_copy(
            src_ref=comm.at[g],
            dst_ref=bcomm.at[g],
            send_sem=b_send.at[g],
            recv_sem=b_recv.at[g],
            device_id=(my_x, 1 - my_y),
            device_id_type=pl.DeviceIdType.MESH,
        )

    def store_cp(g):
        r, c = chunk_slice(g, row0)
        return pltpu.make_async_copy(
            red32.at[g % 3], out_hbm.at[r, c], st_sems.at[g]
        )

    def store2_cp(g):
        r, c = chunk_slice(g, nbr_row0)
        return pltpu.make_async_copy(
            p_buf.at[g % 3], out_hbm.at[r, c], st2_sems.at[g]
        )

    def reduce_forward(g):
        rdma_a(g).wait_recv()
        if g >= 3:
            store_cp(g - 3).wait()
        red32[g % 3] = comm[g].astype(jnp.float32) + p_buf[g % 3]
        store_cp(g).start()
        comm[g] = red32[g % 3].astype(jnp.bfloat16)
        rdma_b(g).start()

    for g in range(_NG):
        c, p = g // _PANELS, g % _PANELS
        if p == 0:
            if c + 1 < _NB:
                w_cp(c + 1).start()
            w_cp(c).wait()
        if g < _PANELS:
            dy_cp(g).wait()
        if g >= 3:
            rdma_a(g - 3).wait_send()
        p_buf[g % 3] = lax.dot_general(
            dy_v[pl.ds(p * pm, pm), :], w_buf[c % 2],
            (((1,), (1,)), ((), ())),
            preferred_element_type=jnp.float32,
        )
        pb16[g % 3] = p_buf[g % 3].astype(jnp.bfloat16)
        rdma_a(g).start()
        if g >= 2:
            reduce_forward(g - 2)
    reduce_forward(_NG - 2)
    reduce_forward(_NG - 1)

    for g in range(_NG - 3, _NG):
        rdma_a(g).wait_send()

    for g in range(_NG):
        rdma_b(g).wait()
        if g >= 3:
            store2_cp(g - 3).wait()
        p_buf[g % 3] = bcomm[g].astype(jnp.float32)
        store2_cp(g).start()

    for g in range(_NG - 3, _NG):
        store_cp(g).wait()
        store2_cp(g).wait()


def kernel(dy, W):
    m, k = dy.shape
    n = W.shape[0]
    band = m // 2
    pm = band // _PANELS
    cb = n // _NB

    return pl.pallas_call(
        _fused_body,
        out_shape=jax.ShapeDtypeStruct((m, n), jnp.float32),
        in_specs=[
            pl.BlockSpec(memory_space=pl.ANY),
            pl.BlockSpec(memory_space=pl.ANY),
        ],
        out_specs=pl.BlockSpec(memory_space=pl.ANY),
        scratch_shapes=[
            pltpu.VMEM((band, k), jnp.float32),
            pltpu.VMEM((2, cb, k), jnp.float32),
            pltpu.VMEM((3, pm, cb), jnp.float32),
            pltpu.VMEM((3, pm, cb), jnp.bfloat16),
            pltpu.VMEM((3, pm, cb), jnp.float32),
            pltpu.VMEM((_NG, pm, cb), jnp.bfloat16),
            pltpu.VMEM((_NG, pm, cb), jnp.bfloat16),
            pltpu.SemaphoreType.DMA((4,)),
            pltpu.SemaphoreType.DMA((_NG,)),
            pltpu.SemaphoreType.DMA((_NG,)),
            pltpu.SemaphoreType.DMA((3,)),
            pltpu.SemaphoreType.DMA((_NG,)),
            pltpu.SemaphoreType.DMA((_NG,)),
            pltpu.SemaphoreType.DMA((_NG,)),
        ],
        compiler_params=pltpu.CompilerParams(
            collective_id=0,
            vmem_limit_bytes=100 * 1024 * 1024,
        ),
    )(dy, W)
